# baseline (speedup 1.0000x reference)
import functools

import jax
import jax.numpy as jnp
from jax import lax
from jax.experimental import pallas as pl
from jax.experimental.pallas import tpu as pltpu
from jax.experimental.pallas import tpu_sc as plsc

NC = 2
NS = 16
NW = NC * NS
L = 16
K = 128


def _sc_mesh():
  return plsc.VectorSubcoreMesh(core_axis_name="c", subcore_axis_name="s")


def kernel(X, edge_index, W, b):
  N, Din = X.shape
  Dout = W.shape[1]
  E = edge_index.shape[1]

  row_unit = NS * K
  NPAD = ((N + 1 + row_unit - 1) // row_unit) * row_unit
  TPR = NPAD // NS

  C = (((E + NW * K - 1) // (NW * K)) + 3) // 4 * 4
  EPW = C * K
  EPAD = EPW * NW
  H = 2
  CH = C // H

  ei = edge_index.astype(jnp.int32)
  last = (NW - 1) * EPW
  pad = N + jnp.arange(EPAD - E, dtype=jnp.int32) % (NPAD - N)
  padb = jnp.broadcast_to(pad, (2, EPAD - E))
  tail = jnp.concatenate([ei[:, last:], padb], axis=1)

  @functools.partial(
      pl.kernel,
      out_type=jax.ShapeDtypeStruct((NC, NPAD), jnp.float32),
      mesh=_sc_mesh(),
      scratch_types=[
          pltpu.VMEM((EPW,), jnp.int32),
          pltpu.VMEM((K,), jnp.float32),
          pltpu.VMEM((TPR,), jnp.float32),
          pltpu.VMEM_SHARED((NPAD,), jnp.float32),
          pltpu.SemaphoreType.DMA,
      ],
  )
  def deg_kernel(ei_hbm, tail_hbm, out_hbm, idx_v, ones_v, zero_v, deg_sp,
                 sem):
    c = lax.axis_index("c")
    s = lax.axis_index("s")
    wid = c * NS + s
    @pl.when(wid < NW - 1)
    def _():
      pltpu.sync_copy(ei_hbm.at[1, pl.ds(wid * EPW, EPW)], idx_v)

    @pl.when(wid == NW - 1)
    def _():
      pltpu.sync_copy(tail_hbm.at[1], idx_v)

    for i in range(K // L):
      ones_v[pl.ds(i * L, L)] = jnp.ones((L,), jnp.float32)
    for i in range(TPR // L):
      zero_v[pl.ds(i * L, L)] = jnp.zeros((L,), jnp.float32)
    pltpu.sync_copy(zero_v, deg_sp.at[pl.ds(s * TPR, TPR)])
    plsc.subcore_barrier()

    def body(j, carry):
      pltpu.async_copy(ones_v, deg_sp.at[idx_v.at[pl.ds(j * K, K)]], sem,
                       add=True)
      return carry

    lax.fori_loop(0, C, body, 0)
    pltpu.make_async_copy(tail_hbm.at[1], idx_v, sem).wait()
    plsc.subcore_barrier()
    pltpu.sync_copy(deg_sp.at[pl.ds(s * TPR, TPR)],
                    out_hbm.at[c, pl.ds(s * TPR, TPR)])

  degp = deg_kernel(ei, tail)

  MB = 2048

  def mm_body(x_ref, w_ref, b_ref, dg_ref, g_ref):
    d = dg_ref[0:1, :] + dg_ref[1:2, :] + 1.0
    dinv = jnp.transpose(lax.rsqrt(jnp.maximum(d, 1.0)), (1, 0))
    h = jnp.dot(x_ref[...], w_ref[...],
                preferred_element_type=jnp.float32) + b_ref[...]
    g_ref[...] = h * dinv

  g = pl.pallas_call(
      mm_body,
      grid=((N + MB - 1) // MB,),
      in_specs=[
          pl.BlockSpec((MB, Din), lambda i: (i, 0)),
          pl.BlockSpec((Din, Dout), lambda i: (0, 0)),
          pl.BlockSpec((1, Dout), lambda i: (0, 0)),
          pl.BlockSpec((NC, MB), lambda i: (0, i)),
      ],
      out_specs=pl.BlockSpec((MB, Dout), lambda i: (i, 0)),
      out_shape=jax.ShapeDtypeStruct((NPAD, Dout), jnp.float32),
  )(X, W, b.reshape(1, Dout), degp)

  @functools.partial(
      pl.kernel,
      out_type=jax.ShapeDtypeStruct((NC, NPAD, Dout), jnp.float32),
      mesh=_sc_mesh(),
      scratch_types=[
          pltpu.VMEM((CH * K,), jnp.int32),
          pltpu.VMEM((CH * K,), jnp.int32),
          pltpu.VMEM((K, Dout), jnp.float32),
          pltpu.VMEM((K, Dout), jnp.float32),
          pltpu.VMEM_SHARED((NPAD, Dout), jnp.float32),
          pltpu.SemaphoreType.DMA,
          pltpu.SemaphoreType.DMA,
      ],
  )
  def scat_kernel(g_hbm, ei_hbm, tail_hbm, out_hbm,
                  si_v, di_v, rows_a, rows_b, acc_sp, sem_a, sem_b):
    c = lax.axis_index("c")
    s = lax.axis_index("s")
    wid = c * NS + s

    def zbody(i, carry):
      for jj in range(Dout // L):
        rows_a[i, pl.ds(jj * L, L)] = jnp.zeros((L,), jnp.float32)
      return carry

    lax.fori_loop(0, K, zbody, 0)
    for r in range(TPR // K):
      pltpu.sync_copy(rows_a, acc_sp.at[pl.ds(s * TPR + r * K, K)])

    def load_idx(h):
      @pl.when(wid < NW - 1)
      def _():
        pltpu.sync_copy(ei_hbm.at[0, pl.ds(wid * EPW + h * CH * K, CH * K)],
                        si_v)
        pltpu.sync_copy(ei_hbm.at[1, pl.ds(wid * EPW + h * CH * K, CH * K)],
                        di_v)

      @pl.when(wid == NW - 1)
      def _():
        pltpu.sync_copy(tail_hbm.at[0, pl.ds(h * CH * K, CH * K)], si_v)
        pltpu.sync_copy(tail_hbm.at[1, pl.ds(h * CH * K, CH * K)], di_v)

    def prime():
      pltpu.async_copy(g_hbm.at[si_v.at[pl.ds(0, K)]], rows_a, sem_a)
      pltpu.async_copy(g_hbm.at[si_v.at[pl.ds(K, K)]], rows_b, sem_b)

    load_idx(0)
    prime()
    plsc.subcore_barrier()

    for h in range(H):
      if h > 0:
        load_idx(h)
        prime()

      def body(jj, carry):
        j = jj * 2
        for rows_v, sem, off in ((rows_a, sem_a, 0), (rows_b, sem_b, 1)):
          pltpu.make_async_copy(g_hbm.at[si_v.at[pl.ds((j + off) * K, K)]],
                                rows_v, sem).wait()
          pltpu.sync_copy(rows_v, acc_sp.at[di_v.at[pl.ds((j + off) * K, K)]],
                          add=True)
          nxt = jnp.minimum((j + off + 2) * K, (CH - 1) * K)
          pltpu.async_copy(g_hbm.at[si_v.at[pl.ds(nxt, K)]], rows_v, sem)
        return carry

      lax.fori_loop(0, CH // 2, body, 0)
      pltpu.make_async_copy(g_hbm.at[si_v.at[pl.ds(0, K)]], rows_a, sem_a).wait()
      pltpu.make_async_copy(g_hbm.at[si_v.at[pl.ds(0, K)]], rows_b, sem_b).wait()
    plsc.subcore_barrier()
    for r in range(TPR // K):
      pltpu.sync_copy(acc_sp.at[pl.ds(s * TPR + r * K, K)],
                      out_hbm.at[c, pl.ds(s * TPR + r * K, K)])

  accp = scat_kernel(g, ei, tail)

  MB2 = 2048

  def fin_body(a0_ref, a1_ref, g_ref, dg_ref, o_ref):
    d = dg_ref[0:1, :] + dg_ref[1:2, :] + 1.0
    dinv = jnp.transpose(lax.rsqrt(jnp.maximum(d, 1.0)), (1, 0))
    tot = a0_ref[0] + a1_ref[0] + g_ref[...]
    o_ref[...] = jnp.maximum(tot * dinv, 0.0)

  out = pl.pallas_call(
      fin_body,
      grid=((N + MB2 - 1) // MB2,),
      in_specs=[
          pl.BlockSpec((1, MB2, Dout), lambda i: (0, i, 0)),
          pl.BlockSpec((1, MB2, Dout), lambda i: (1, i, 0)),
          pl.BlockSpec((MB2, Dout), lambda i: (i, 0)),
          pl.BlockSpec((NC, MB2), lambda i: (0, i)),
      ],
      out_specs=pl.BlockSpec((MB2, Dout), lambda i: (i, 0)),
      out_shape=jax.ShapeDtypeStruct((N, Dout), jnp.float32),
  )(accp, accp, g, degp)

  return out


# speedup vs baseline: 1.0018x; 1.0018x over previous
import functools

import jax
import jax.numpy as jnp
from jax import lax
from jax.experimental import pallas as pl
from jax.experimental.pallas import tpu as pltpu
from jax.experimental.pallas import tpu_sc as plsc

NC = 2
NS = 16
NW = NC * NS
L = 16
K = 128


def _sc_mesh():
  return plsc.VectorSubcoreMesh(core_axis_name="c", subcore_axis_name="s")


def kernel(X, edge_index, W, b):
  N, Din = X.shape
  Dout = W.shape[1]
  E = edge_index.shape[1]

  row_unit = NS * K
  NPAD = ((N + 1 + row_unit - 1) // row_unit) * row_unit
  TPR = NPAD // NS

  C = (((E + NW * K - 1) // (NW * K)) + 3) // 4 * 4
  EPW = C * K
  EPAD = EPW * NW
  H = 2
  CH = C // H

  ei = edge_index.astype(jnp.int32)
  last = (NW - 1) * EPW
  pad = N + jnp.arange(EPAD - E, dtype=jnp.int32) % (NPAD - N)
  padb = jnp.broadcast_to(pad, (2, EPAD - E))
  tail = jnp.concatenate([ei[:, last:], padb], axis=1)

  @functools.partial(
      pl.kernel,
      out_type=jax.ShapeDtypeStruct((NC, NPAD), jnp.float32),
      mesh=_sc_mesh(),
      scratch_types=[
          pltpu.VMEM((EPW,), jnp.int32),
          pltpu.VMEM((K,), jnp.float32),
          pltpu.VMEM((TPR,), jnp.float32),
          pltpu.VMEM_SHARED((NPAD,), jnp.float32),
          pltpu.SemaphoreType.DMA,
      ],
  )
  def deg_kernel(ei_hbm, tail_hbm, out_hbm, idx_v, ones_v, zero_v, deg_sp,
                 sem):
    c = lax.axis_index("c")
    s = lax.axis_index("s")
    wid = c * NS + s
    @pl.when(wid < NW - 1)
    def _():
      pltpu.sync_copy(ei_hbm.at[1, pl.ds(wid * EPW, EPW)], idx_v)

    @pl.when(wid == NW - 1)
    def _():
      pltpu.sync_copy(tail_hbm.at[1], idx_v)

    for i in range(K // L):
      ones_v[pl.ds(i * L, L)] = jnp.ones((L,), jnp.float32)
    for i in range(TPR // L):
      zero_v[pl.ds(i * L, L)] = jnp.zeros((L,), jnp.float32)
    pltpu.sync_copy(zero_v, deg_sp.at[pl.ds(s * TPR, TPR)])
    plsc.subcore_barrier()

    def body(j, carry):
      pltpu.async_copy(ones_v, deg_sp.at[idx_v.at[pl.ds(j * K, K)]], sem,
                       add=True)
      return carry

    lax.fori_loop(0, C, body, 0)
    pltpu.make_async_copy(tail_hbm.at[1], idx_v, sem).wait()
    plsc.subcore_barrier()
    pltpu.sync_copy(deg_sp.at[pl.ds(s * TPR, TPR)],
                    out_hbm.at[c, pl.ds(s * TPR, TPR)])

  degp = deg_kernel(ei, tail)

  MB = 2048

  def mm_body(x_ref, w_ref, b_ref, dg_ref, g_ref):
    d = dg_ref[0:1, :] + dg_ref[1:2, :] + 1.0
    dinv = jnp.transpose(lax.rsqrt(jnp.maximum(d, 1.0)), (1, 0))
    h = jnp.dot(x_ref[...], w_ref[...],
                preferred_element_type=jnp.float32) + b_ref[...]
    g_ref[...] = h * dinv

  g = pl.pallas_call(
      mm_body,
      grid=((N + MB - 1) // MB,),
      in_specs=[
          pl.BlockSpec((MB, Din), lambda i: (i, 0)),
          pl.BlockSpec((Din, Dout), lambda i: (0, 0)),
          pl.BlockSpec((1, Dout), lambda i: (0, 0)),
          pl.BlockSpec((NC, MB), lambda i: (0, i)),
      ],
      out_specs=pl.BlockSpec((MB, Dout), lambda i: (i, 0)),
      out_shape=jax.ShapeDtypeStruct((NPAD, Dout), jnp.float32),
  )(X, W, b.reshape(1, Dout), degp)

  @functools.partial(
      pl.kernel,
      out_type=jax.ShapeDtypeStruct((NC, NPAD, Dout), jnp.float32),
      mesh=_sc_mesh(),
      scratch_types=[
          pltpu.VMEM((CH * K,), jnp.int32),
          pltpu.VMEM((CH * K,), jnp.int32),
          pltpu.VMEM((K, Dout), jnp.float32),
          pltpu.VMEM((K, Dout), jnp.float32),
          pltpu.VMEM_SHARED((NPAD, Dout), jnp.float32),
          pltpu.SemaphoreType.DMA,
          pltpu.SemaphoreType.DMA,
      ],
  )
  def scat_kernel(g_hbm, ei_hbm, tail_hbm, out_hbm,
                  si_v, di_v, rows_a, rows_b, acc_sp, sem_a, sem_b):
    c = lax.axis_index("c")
    s = lax.axis_index("s")
    wid = c * NS + s

    def zbody(i, carry):
      for jj in range(Dout // L):
        rows_a[i, pl.ds(jj * L, L)] = jnp.zeros((L,), jnp.float32)
      return carry

    lax.fori_loop(0, K, zbody, 0)
    for r in range(TPR // K):
      pltpu.sync_copy(rows_a, acc_sp.at[pl.ds(s * TPR + r * K, K)])
    plsc.subcore_barrier()

    for h in range(H):
      @pl.when(wid < NW - 1)
      def _(h=h):
        pltpu.sync_copy(ei_hbm.at[0, pl.ds(wid * EPW + h * CH * K, CH * K)],
                        si_v)
        pltpu.sync_copy(ei_hbm.at[1, pl.ds(wid * EPW + h * CH * K, CH * K)],
                        di_v)

      @pl.when(wid == NW - 1)
      def _(h=h):
        pltpu.sync_copy(tail_hbm.at[0, pl.ds(h * CH * K, CH * K)], si_v)
        pltpu.sync_copy(tail_hbm.at[1, pl.ds(h * CH * K, CH * K)], di_v)

      pltpu.async_copy(g_hbm.at[si_v.at[pl.ds(0, K)]], rows_a, sem_a)
      pltpu.async_copy(g_hbm.at[si_v.at[pl.ds(K, K)]], rows_b, sem_b)

      def body(jj, carry):
        j = jj * 2
        for rows_v, sem, off in ((rows_a, sem_a, 0), (rows_b, sem_b, 1)):
          pltpu.make_async_copy(g_hbm.at[si_v.at[pl.ds((j + off) * K, K)]],
                                rows_v, sem).wait()
          pltpu.sync_copy(rows_v, acc_sp.at[di_v.at[pl.ds((j + off) * K, K)]],
                          add=True)
          nxt = jnp.minimum((j + off + 2) * K, (CH - 1) * K)
          pltpu.async_copy(g_hbm.at[si_v.at[pl.ds(nxt, K)]], rows_v, sem)
        return carry

      lax.fori_loop(0, CH // 2, body, 0)
      pltpu.make_async_copy(g_hbm.at[si_v.at[pl.ds(0, K)]], rows_a, sem_a).wait()
      pltpu.make_async_copy(g_hbm.at[si_v.at[pl.ds(0, K)]], rows_b, sem_b).wait()
    plsc.subcore_barrier()
    for r in range(TPR // K):
      pltpu.sync_copy(acc_sp.at[pl.ds(s * TPR + r * K, K)],
                      out_hbm.at[c, pl.ds(s * TPR + r * K, K)])

  accp = scat_kernel(g, ei, tail)

  MB2 = 2048

  def fin_body(a0_ref, a1_ref, g_ref, dg_ref, o_ref):
    d = dg_ref[0:1, :] + dg_ref[1:2, :] + 1.0
    dinv = jnp.transpose(lax.rsqrt(jnp.maximum(d, 1.0)), (1, 0))
    tot = a0_ref[0] + a1_ref[0] + g_ref[...]
    o_ref[...] = jnp.maximum(tot * dinv, 0.0)

  out = pl.pallas_call(
      fin_body,
      grid=((N + MB2 - 1) // MB2,),
      in_specs=[
          pl.BlockSpec((1, MB2, Dout), lambda i: (0, i, 0)),
          pl.BlockSpec((1, MB2, Dout), lambda i: (1, i, 0)),
          pl.BlockSpec((MB2, Dout), lambda i: (i, 0)),
          pl.BlockSpec((NC, MB2), lambda i: (0, i)),
      ],
      out_specs=pl.BlockSpec((MB2, Dout), lambda i: (i, 0)),
      out_shape=jax.ShapeDtypeStruct((N, Dout), jnp.float32),
  )(accp, accp, g, degp)

  return out
